# baseline (speedup 1.0000x reference)
import functools

import jax
import jax.numpy as jnp
from jax import lax
from jax.experimental import pallas as pl
from jax.experimental.pallas import tpu as pltpu
from jax.experimental.pallas import tpu_sc as plsc

MOD_VOCAB = 11
OUT_DIM = 80
_TPAD = 304

_CHUNK = 128
_LANES = 16


def _build_sc_call(n_batch: int, n_len: int):
    info = plsc.get_sparse_core_info()
    nc, ns = info.num_cores, info.num_subcores
    nw = nc * ns
    tc_per_l = n_batch // _CHUNK
    n_pairs = n_len * tc_per_l
    ppw = n_pairs // nw
    dg_n = OUT_DIM // 8
    out_rows = n_len * OUT_DIM * n_batch // _CHUNK
    nbuf = 2
    assert ppw * nw == n_pairs and ppw % nbuf == 0

    mesh = plsc.VectorSubcoreMesh(core_axis_name="c", subcore_axis_name="s")

    @functools.partial(
        pl.kernel,
        mesh=mesh,
        compiler_params=pltpu.CompilerParams(
            use_tc_tiling_on_sc=False, needs_layout_passes=False),
        out_type=jax.ShapeDtypeStruct((out_rows, _CHUNK), jnp.float32),
        scratch_types=[
            pltpu.VMEM((ppw, _CHUNK), jnp.int32),
            pltpu.VMEM((ppw, _CHUNK), jnp.int32),
            pltpu.VMEM((OUT_DIM * _TPAD,), jnp.float32),
            [pltpu.VMEM((OUT_DIM, _CHUNK), jnp.float32) for _ in range(nbuf)],
            [pltpu.SemaphoreType.DMA for _ in range(nbuf)],
        ],
    )
    def enc(seqt_hbm, modt_hbm, table_hbm, out_hbm, idx_v, mod_v, table_v,
            bufs, wsems):
        wid = lax.axis_index("s") * nc + lax.axis_index("c")
        p0 = wid * ppw

        pltpu.sync_copy(table_hbm, table_v)
        pltpu.sync_copy(seqt_hbm.at[pl.ds(p0, ppw)], idx_v)
        pltpu.sync_copy(modt_hbm.at[pl.ds(p0, ppw)], mod_v)

        def comb_body(jj, _):
            for k in range(_CHUNK // _LANES):
                sl = pl.ds(k * _LANES, _LANES)
                idx_v[jj, sl] = idx_v[jj, sl] * MOD_VOCAB + mod_v[jj, sl]
            return 0

        lax.fori_loop(0, ppw, comb_body, 0)

        def drain(bi):
            for dg in range(dg_n):
                pltpu.make_async_copy(
                    bufs[bi].at[pl.ds(dg * 8, 8)],
                    out_hbm.at[pl.ds(0, 8)], wsems[bi]).wait()

        def outer(t, _):
            for bi in range(nbuf):
                j = t * nbuf + bi
                p = p0 + j
                il = p // tc_per_l
                tc = p - il * tc_per_l

                @pl.when(t > 0)
                def _():
                    drain(bi)

                chunks = [
                    idx_v[j, pl.ds(k * _LANES, _LANES)]
                    for k in range(_CHUNK // _LANES)
                ]

                def dg_body(dg, _):
                    for dd0 in range(0, 8, 4):
                        vals = []
                        for dd in range(dd0, dd0 + 4):
                            d = dg * 8 + dd
                            tv = table_v.at[pl.ds(d * _TPAD, _TPAD)]
                            vals.append([
                                plsc.load_gather(tv, [chunks[k]])
                                for k in range(_CHUNK // _LANES)
                            ])
                        for i, dd in enumerate(range(dd0, dd0 + 4)):
                            d = dg * 8 + dd
                            for k in range(_CHUNK // _LANES):
                                bufs[bi][d, pl.ds(k * _LANES, _LANES)] = (
                                    vals[i][k])
                    row0 = (il * dg_n + dg) * (tc_per_l * 8) + tc * 8
                    pltpu.async_copy(
                        bufs[bi].at[pl.ds(dg * 8, 8)],
                        out_hbm.at[pl.ds(row0, 8)], wsems[bi])
                    return 0

                lax.fori_loop(0, dg_n, dg_body, 0)
            return 0

        lax.fori_loop(0, ppw // nbuf, outer, 0)
        for bi in range(nbuf):
            drain(bi)

    return enc


def kernel(sequence, modifications, seq_table, mod_table):
    bsz, slen = sequence.shape
    aa_vocab = seq_table.shape[0]

    comb_table = jnp.concatenate(
        [
            jnp.repeat(seq_table, MOD_VOCAB, axis=0),
            jnp.tile(mod_table, (aa_vocab, 1)),
        ],
        axis=1,
    )
    n_comb = comb_table.shape[0]
    tablet = jnp.pad(comb_table.T, ((0, 0), (0, _TPAD - n_comb))).reshape(-1)

    seqt = sequence.T.reshape(slen * bsz // _CHUNK, _CHUNK).astype(jnp.int32)
    modt = modifications.T.reshape(
        slen * bsz // _CHUNK, _CHUNK).astype(jnp.int32)

    out2d = _build_sc_call(bsz, slen)(seqt, modt, tablet)

    out5 = out2d.reshape(slen, OUT_DIM // 8, bsz // _CHUNK, 8, _CHUNK)
    return out5.transpose(2, 4, 0, 1, 3).reshape(bsz, slen, OUT_DIM)


# speedup vs baseline: 1.0382x; 1.0382x over previous
import functools

import jax
import jax.numpy as jnp
from jax import lax
from jax.experimental import pallas as pl
from jax.experimental.pallas import tpu as pltpu
from jax.experimental.pallas import tpu_sc as plsc

MOD_VOCAB = 11
OUT_DIM = 80
_TPAD = 304

_CHUNK = 128
_LANES = 16


def _build_sc_call(n_batch: int, n_len: int):
    info = plsc.get_sparse_core_info()
    nc, ns = info.num_cores, info.num_subcores
    nw = nc * ns
    tc_per_l = n_batch // _CHUNK
    n_pairs = n_len * tc_per_l
    ppw = n_pairs // nw
    dg_n = OUT_DIM // 8
    out_rows = n_len * OUT_DIM * n_batch // _CHUNK
    nbuf = 2
    assert ppw * nw == n_pairs and ppw % nbuf == 0

    mesh = plsc.VectorSubcoreMesh(core_axis_name="c", subcore_axis_name="s")

    @functools.partial(
        pl.kernel,
        mesh=mesh,
        compiler_params=pltpu.CompilerParams(
            use_tc_tiling_on_sc=False, needs_layout_passes=False),
        out_type=jax.ShapeDtypeStruct((out_rows, _CHUNK), jnp.float32),
        scratch_types=[
            pltpu.VMEM((ppw, _CHUNK), jnp.int32),
            pltpu.VMEM((ppw, _CHUNK), jnp.int32),
            pltpu.VMEM((OUT_DIM * _TPAD,), jnp.float32),
            [pltpu.VMEM((OUT_DIM, _CHUNK), jnp.float32) for _ in range(nbuf)],
            [pltpu.SemaphoreType.DMA for _ in range(nbuf)],
        ],
    )
    def enc(seqt_hbm, modt_hbm, table_hbm, out_hbm, idx_v, mod_v, table_v,
            bufs, wsems):
        wid = lax.axis_index("s") * nc + lax.axis_index("c")
        p0 = wid * ppw

        pltpu.sync_copy(table_hbm, table_v)
        pltpu.sync_copy(seqt_hbm.at[pl.ds(p0, ppw)], idx_v)
        pltpu.sync_copy(modt_hbm.at[pl.ds(p0, ppw)], mod_v)

        def comb_body(jj, _):
            for k in range(_CHUNK // _LANES):
                sl = pl.ds(k * _LANES, _LANES)
                idx_v[jj, sl] = idx_v[jj, sl] * MOD_VOCAB + mod_v[jj, sl]
            return 0

        lax.fori_loop(0, ppw, comb_body, 0)

        def drain(bi):
            for dg in range(dg_n):
                pltpu.make_async_copy(
                    bufs[bi].at[pl.ds(dg * 8, 8)],
                    out_hbm.at[pl.ds(0, 8)], wsems[bi]).wait()

        def outer(t, _):
            for bi in range(nbuf):
                j = t * nbuf + bi
                p = p0 + j
                il = p // tc_per_l
                tc = p - il * tc_per_l

                @pl.when(t > 0)
                def _():
                    drain(bi)

                chunks = [
                    idx_v[j, pl.ds(k * _LANES, _LANES)]
                    for k in range(_CHUNK // _LANES)
                ]

                def dg_body(dg, _):
                    for dd in range(8):
                        d = dg * 8 + dd
                        tv = table_v.at[pl.ds(d * _TPAD, _TPAD)]
                        vals = [
                            plsc.load_gather(tv, [chunks[k]])
                            for k in range(_CHUNK // _LANES)
                        ]
                        for k in range(_CHUNK // _LANES):
                            bufs[bi][d, pl.ds(k * _LANES, _LANES)] = vals[k]
                    row0 = (il * dg_n + dg) * (tc_per_l * 8) + tc * 8
                    pltpu.async_copy(
                        bufs[bi].at[pl.ds(dg * 8, 8)],
                        out_hbm.at[pl.ds(row0, 8)], wsems[bi])
                    return 0

                lax.fori_loop(0, dg_n, dg_body, 0)
            return 0

        lax.fori_loop(0, ppw // nbuf, outer, 0)
        for bi in range(nbuf):
            drain(bi)

    return enc


def kernel(sequence, modifications, seq_table, mod_table):
    bsz, slen = sequence.shape
    aa_vocab = seq_table.shape[0]

    comb_table = jnp.concatenate(
        [
            jnp.repeat(seq_table, MOD_VOCAB, axis=0),
            jnp.tile(mod_table, (aa_vocab, 1)),
        ],
        axis=1,
    )
    n_comb = comb_table.shape[0]
    tablet = jnp.pad(comb_table.T, ((0, 0), (0, _TPAD - n_comb))).reshape(-1)

    seqt = sequence.T.reshape(slen * bsz // _CHUNK, _CHUNK).astype(jnp.int32)
    modt = modifications.T.reshape(
        slen * bsz // _CHUNK, _CHUNK).astype(jnp.int32)

    out2d = _build_sc_call(bsz, slen)(seqt, modt, tablet)

    out5 = out2d.reshape(slen, OUT_DIM // 8, bsz // _CHUNK, 8, _CHUNK)
    return out5.transpose(2, 4, 0, 1, 3).reshape(bsz, slen, OUT_DIM)


# speedup vs baseline: 1.0530x; 1.0142x over previous
import functools

import jax
import jax.numpy as jnp
from jax import lax
from jax.experimental import pallas as pl
from jax.experimental.pallas import tpu as pltpu
from jax.experimental.pallas import tpu_sc as plsc

MOD_VOCAB = 11
OUT_DIM = 80
_TPAD = 304

_CHUNK = 128
_LANES = 16


def _build_sc_call(n_batch: int, n_len: int):
    info = plsc.get_sparse_core_info()
    nc, ns = info.num_cores, info.num_subcores
    nw = nc * ns
    tc_per_l = n_batch // _CHUNK
    n_pairs = n_len * tc_per_l
    ppw = n_pairs // nw
    dg_n = OUT_DIM // 8
    out_rows = n_len * OUT_DIM * n_batch // _CHUNK
    nbuf = 2
    assert ppw * nw == n_pairs and ppw % nbuf == 0

    mesh = plsc.VectorSubcoreMesh(core_axis_name="c", subcore_axis_name="s")

    @functools.partial(
        pl.kernel,
        mesh=mesh,
        compiler_params=pltpu.CompilerParams(
            use_tc_tiling_on_sc=False, needs_layout_passes=False),
        out_type=jax.ShapeDtypeStruct((out_rows, _CHUNK), jnp.float32),
        scratch_types=[
            pltpu.VMEM((ppw, _CHUNK), jnp.int32),
            pltpu.VMEM((ppw, _CHUNK), jnp.int32),
            pltpu.VMEM((OUT_DIM * _TPAD,), jnp.float32),
            [pltpu.VMEM((OUT_DIM, _CHUNK), jnp.float32) for _ in range(nbuf)],
            [pltpu.SemaphoreType.DMA for _ in range(nbuf)],
        ],
    )
    def enc(seqt_hbm, modt_hbm, table_hbm, out_hbm, idx_v, mod_v, table_v,
            bufs, wsems):
        wid = lax.axis_index("s") * nc + lax.axis_index("c")
        p0 = wid * ppw

        pltpu.sync_copy(table_hbm, table_v)
        pltpu.sync_copy(seqt_hbm.at[pl.ds(p0, ppw)], idx_v)
        pltpu.sync_copy(modt_hbm.at[pl.ds(p0, ppw)], mod_v)

        def comb_body(jj, _):
            for k in range(_CHUNK // _LANES):
                sl = pl.ds(k * _LANES, _LANES)
                idx_v[jj, sl] = idx_v[jj, sl] * MOD_VOCAB + mod_v[jj, sl]
            return 0

        lax.fori_loop(0, ppw, comb_body, 0)

        def drain(bi):
            for dg in range(dg_n):
                pltpu.make_async_copy(
                    bufs[bi].at[pl.ds(dg * 8, 8)],
                    out_hbm.at[pl.ds(0, 8)], wsems[bi]).wait()

        def outer(t, _):
            for bi in range(nbuf):
                j = t * nbuf + bi
                p = p0 + j
                il = p // tc_per_l
                tc = p - il * tc_per_l

                @pl.when(t > 0)
                def _():
                    drain(bi)

                chunks = [
                    idx_v[j, pl.ds(k * _LANES, _LANES)]
                    for k in range(_CHUNK // _LANES)
                ]

                def dg_body(dg, _):
                    def loads(d):
                        tv = table_v.at[pl.ds(d * _TPAD, _TPAD)]
                        return [
                            plsc.load_gather(tv, [chunks[k]])
                            for k in range(_CHUNK // _LANES)
                        ]

                    def stores(d, vals):
                        for k in range(_CHUNK // _LANES):
                            bufs[bi][d, pl.ds(k * _LANES, _LANES)] = vals[k]

                    prev = loads(dg * 8)
                    for dd in range(1, 8):
                        cur = loads(dg * 8 + dd)
                        stores(dg * 8 + dd - 1, prev)
                        prev = cur
                    stores(dg * 8 + 7, prev)
                    row0 = (il * dg_n + dg) * (tc_per_l * 8) + tc * 8
                    pltpu.async_copy(
                        bufs[bi].at[pl.ds(dg * 8, 8)],
                        out_hbm.at[pl.ds(row0, 8)], wsems[bi])
                    return 0

                lax.fori_loop(0, dg_n, dg_body, 0)
            return 0

        lax.fori_loop(0, ppw // nbuf, outer, 0)
        for bi in range(nbuf):
            drain(bi)

    return enc


def kernel(sequence, modifications, seq_table, mod_table):
    bsz, slen = sequence.shape
    aa_vocab = seq_table.shape[0]

    comb_table = jnp.concatenate(
        [
            jnp.repeat(seq_table, MOD_VOCAB, axis=0),
            jnp.tile(mod_table, (aa_vocab, 1)),
        ],
        axis=1,
    )
    n_comb = comb_table.shape[0]
    tablet = jnp.pad(comb_table.T, ((0, 0), (0, _TPAD - n_comb))).reshape(-1)

    seqt = sequence.T.reshape(slen * bsz // _CHUNK, _CHUNK).astype(jnp.int32)
    modt = modifications.T.reshape(
        slen * bsz // _CHUNK, _CHUNK).astype(jnp.int32)

    out2d = _build_sc_call(bsz, slen)(seqt, modt, tablet)

    out5 = out2d.reshape(slen, OUT_DIM // 8, bsz // _CHUNK, 8, _CHUNK)
    return out5.transpose(2, 4, 0, 1, 3).reshape(bsz, slen, OUT_DIM)


# speedup vs baseline: 1.0534x; 1.0004x over previous
import functools

import jax
import jax.numpy as jnp
from jax import lax
from jax.experimental import pallas as pl
from jax.experimental.pallas import tpu as pltpu
from jax.experimental.pallas import tpu_sc as plsc

MOD_VOCAB = 11
OUT_DIM = 80
_TPAD = 304

_CHUNK = 128
_LANES = 16


def _build_sc_call(n_batch: int, n_len: int):
    info = plsc.get_sparse_core_info()
    nc, ns = info.num_cores, info.num_subcores
    nw = nc * ns
    tc_per_l = n_batch // _CHUNK
    n_pairs = n_len * tc_per_l
    ppw = n_pairs // nw
    dg_n = OUT_DIM // 8
    out_rows = n_len * OUT_DIM * n_batch // _CHUNK
    nbuf = 4
    assert ppw * nw == n_pairs and ppw % nbuf == 0

    mesh = plsc.VectorSubcoreMesh(core_axis_name="c", subcore_axis_name="s")

    @functools.partial(
        pl.kernel,
        mesh=mesh,
        compiler_params=pltpu.CompilerParams(
            use_tc_tiling_on_sc=False, needs_layout_passes=False),
        out_type=jax.ShapeDtypeStruct((out_rows, _CHUNK), jnp.float32),
        scratch_types=[
            pltpu.VMEM((ppw, _CHUNK), jnp.int32),
            pltpu.VMEM((ppw, _CHUNK), jnp.int32),
            pltpu.VMEM((OUT_DIM * _TPAD,), jnp.float32),
            [pltpu.VMEM((OUT_DIM, _CHUNK), jnp.float32) for _ in range(nbuf)],
            [pltpu.SemaphoreType.DMA for _ in range(nbuf)],
        ],
    )
    def enc(seqt_hbm, modt_hbm, table_hbm, out_hbm, idx_v, mod_v, table_v,
            bufs, wsems):
        wid = lax.axis_index("s") * nc + lax.axis_index("c")
        p0 = wid * ppw

        pltpu.sync_copy(table_hbm, table_v)
        pltpu.sync_copy(seqt_hbm.at[pl.ds(p0, ppw)], idx_v)
        pltpu.sync_copy(modt_hbm.at[pl.ds(p0, ppw)], mod_v)

        def comb_body(jj, _):
            for k in range(_CHUNK // _LANES):
                sl = pl.ds(k * _LANES, _LANES)
                idx_v[jj, sl] = idx_v[jj, sl] * MOD_VOCAB + mod_v[jj, sl]
            return 0

        lax.fori_loop(0, ppw, comb_body, 0)

        def drain(bi):
            for dg in range(dg_n):
                pltpu.make_async_copy(
                    bufs[bi].at[pl.ds(dg * 8, 8)],
                    out_hbm.at[pl.ds(0, 8)], wsems[bi]).wait()

        def outer(t, _):
            for bi in range(nbuf):
                j = t * nbuf + bi
                p = p0 + j
                il = p // tc_per_l
                tc = p - il * tc_per_l

                @pl.when(t > 0)
                def _():
                    drain(bi)

                chunks = [
                    idx_v[j, pl.ds(k * _LANES, _LANES)]
                    for k in range(_CHUNK // _LANES)
                ]

                def dg_body(dg, _):
                    def loads(d):
                        tv = table_v.at[pl.ds(d * _TPAD, _TPAD)]
                        return [
                            plsc.load_gather(tv, [chunks[k]])
                            for k in range(_CHUNK // _LANES)
                        ]

                    def stores(d, vals):
                        for k in range(_CHUNK // _LANES):
                            bufs[bi][d, pl.ds(k * _LANES, _LANES)] = vals[k]

                    prev = loads(dg * 8)
                    for dd in range(1, 8):
                        cur = loads(dg * 8 + dd)
                        stores(dg * 8 + dd - 1, prev)
                        prev = cur
                    stores(dg * 8 + 7, prev)
                    row0 = (il * dg_n + dg) * (tc_per_l * 8) + tc * 8
                    pltpu.async_copy(
                        bufs[bi].at[pl.ds(dg * 8, 8)],
                        out_hbm.at[pl.ds(row0, 8)], wsems[bi])
                    return 0

                lax.fori_loop(0, dg_n, dg_body, 0)
            return 0

        lax.fori_loop(0, ppw // nbuf, outer, 0)
        for bi in range(nbuf):
            drain(bi)

    return enc


def kernel(sequence, modifications, seq_table, mod_table):
    bsz, slen = sequence.shape
    aa_vocab = seq_table.shape[0]

    comb_table = jnp.concatenate(
        [
            jnp.repeat(seq_table, MOD_VOCAB, axis=0),
            jnp.tile(mod_table, (aa_vocab, 1)),
        ],
        axis=1,
    )
    n_comb = comb_table.shape[0]
    tablet = jnp.pad(comb_table.T, ((0, 0), (0, _TPAD - n_comb))).reshape(-1)

    seqt = sequence.T.reshape(slen * bsz // _CHUNK, _CHUNK).astype(jnp.int32)
    modt = modifications.T.reshape(
        slen * bsz // _CHUNK, _CHUNK).astype(jnp.int32)

    out2d = _build_sc_call(bsz, slen)(seqt, modt, tablet)

    out5 = out2d.reshape(slen, OUT_DIM // 8, bsz // _CHUNK, 8, _CHUNK)
    return out5.transpose(2, 4, 0, 1, 3).reshape(bsz, slen, OUT_DIM)
